# baseline (speedup 1.0000x reference)
import jax
import jax.numpy as jnp
from jax import lax
from jax.experimental import pallas as pl
from jax.experimental.pallas import tpu as pltpu
from jax.experimental.pallas import tpu_sc as plsc

N = 100000
E = 1600000
NP = 100096
NPAD = NP - N
EP = 1638400
B = 2048
NSC = 2
NT = 16
RPT = NP // NT
DROW = 6272
DLAST = NP - 15 * DROW
CH = 8
NCH = 64 // CH

_mesh = lambda: plsc.VectorSubcoreMesh(core_axis_name="c", subcore_axis_name="s")
_sc_params = lambda: pltpu.CompilerParams(use_tc_tiling_on_sc=False)


def _fill1d(ref, n16, value):
    def body(i, _):
        ref[pl.ds(i * 16, 16)] = jnp.full((16,), value, jnp.float32)
        return 0
    lax.fori_loop(0, n16, body, 0)


def _deg_body(dst_h, z1_h, out0_h, out1_h, dstb_v, ones_v, accum):
    c = lax.axis_index("c")
    s = lax.axis_index("s")
    row0 = s * DROW
    _fill1d(ones_v, B // 16, 1.0)

    @pl.when(s < 15)
    def _():
        pltpu.sync_copy(z1_h.at[pl.ds(row0, DROW)], accum.at[pl.ds(row0, DROW)])

    @pl.when(s == 15)
    def _():
        pltpu.sync_copy(z1_h.at[pl.ds(row0, DLAST)], accum.at[pl.ds(row0, DLAST)])

    plsc.subcore_barrier()
    e0 = (c * NT + s) * (EP // (NSC * NT))
    nb = EP // (NSC * NT) // B

    def body(j, _):
        base = pl.multiple_of(e0 + j * B, 128)
        pltpu.sync_copy(dst_h.at[pl.ds(base, B)], dstb_v)
        pltpu.sync_copy(ones_v, accum.at[dstb_v], add=True)
        return 0

    lax.fori_loop(0, nb, body, 0)
    plsc.subcore_barrier()
    for cc, out_h in ((0, out0_h), (1, out1_h)):
        @pl.when(c == cc)
        def _(out_h=out_h):
            @pl.when(s < 15)
            def _():
                pltpu.sync_copy(accum.at[pl.ds(row0, DROW)],
                                out_h.at[pl.ds(row0, DROW)])

            @pl.when(s == 15)
            def _():
                pltpu.sync_copy(accum.at[pl.ds(row0, DLAST)],
                                out_h.at[pl.ds(row0, DLAST)])


def _sc_degree(dst, z1):
    return pl.kernel(
        _deg_body,
        out_type=(jax.ShapeDtypeStruct((NP,), jnp.float32),
                  jax.ShapeDtypeStruct((NP,), jnp.float32)),
        mesh=_mesh(),
        compiler_params=_sc_params(),
        scratch_types=[
            pltpu.VMEM((B,), jnp.int32),
            pltpu.VMEM((B,), jnp.float32),
            pltpu.VMEM_SHARED((NP,), jnp.float32),
        ],
    )(dst, z1)


def _agg_pass(src_h, dst_h, table_h, zc_h, out_h, idx_v, dstb_v, rows_v,
              accum, e0, n_edges):
    s = lax.axis_index("s")
    row0 = s * RPT
    pltpu.sync_copy(zc_h.at[pl.ds(row0, RPT)], accum.at[pl.ds(row0, RPT)])
    plsc.subcore_barrier()

    def body(j, _):
        base = pl.multiple_of(e0 + j * B, 128)
        pltpu.sync_copy(src_h.at[pl.ds(base, B)], idx_v)
        pltpu.sync_copy(dst_h.at[pl.ds(base, B)], dstb_v)
        pltpu.sync_copy(table_h.at[idx_v], rows_v)
        pltpu.sync_copy(rows_v, accum.at[dstb_v], add=True)
        return 0

    lax.fori_loop(0, n_edges // B, body, 0)
    plsc.subcore_barrier()
    pltpu.sync_copy(accum.at[pl.ds(row0, RPT)], out_h.at[pl.ds(row0, RPT)])


def _agg1_body(src_h, dst_h, t_h, zc_h, out0_h, out1_h, idx_v, dstb_v,
               rows_v, accum):
    c = lax.axis_index("c")
    s = lax.axis_index("s")
    ept = EP // (NSC * NT)
    e0 = (c * NT + s) * ept
    for cc, out_h in ((0, out0_h), (1, out1_h)):
        @pl.when(c == cc)
        def _(out_h=out_h):
            _agg_pass(src_h, dst_h, t_h, zc_h, out_h, idx_v, dstb_v,
                      rows_v, accum, e0, ept)


def _sc_agg1(src, dst, table, zc):
    return pl.kernel(
        _agg1_body,
        out_type=(jax.ShapeDtypeStruct((NP, CH), jnp.float32),
                  jax.ShapeDtypeStruct((NP, CH), jnp.float32)),
        mesh=_mesh(),
        compiler_params=_sc_params(),
        scratch_types=[
            pltpu.VMEM((B,), jnp.int32),
            pltpu.VMEM((B,), jnp.int32),
            pltpu.VMEM((B, CH), jnp.float32),
            pltpu.VMEM_SHARED((NP, CH), jnp.float32),
        ],
    )(src, dst, table, zc)


def _agg2_body(src_h, dst_h, tflat_h, zc_h, out_h, srcb_v, idx_v, dstb_v,
               rows_v, accum):
    c = lax.axis_index("c")
    s = lax.axis_index("s")
    ept = EP // NT
    e0 = s * ept
    row0 = s * RPT
    for q in range(NCH):
        @pl.when(q // (NCH // NSC) == c)
        def _(q=q):
            pltpu.sync_copy(zc_h.at[pl.ds(row0, RPT)],
                            accum.at[pl.ds(row0, RPT)])
            plsc.subcore_barrier()

            def body(j, _):
                base = pl.multiple_of(e0 + j * B, 128)
                pltpu.sync_copy(src_h.at[pl.ds(base, B)], srcb_v)
                pltpu.sync_copy(dst_h.at[pl.ds(base, B)], dstb_v)

                def addq(i, _):
                    idx_v[pl.ds(i * 16, 16)] = (
                        srcb_v[pl.ds(i * 16, 16)] * 16 + q)
                    return 0

                lax.fori_loop(0, B // 16, addq, 0)
                pltpu.sync_copy(tflat_h.at[idx_v], rows_v)
                pltpu.sync_copy(rows_v, accum.at[dstb_v], add=True)
                return 0

            lax.fori_loop(0, ept // B, body, 0)
            plsc.subcore_barrier()
            pltpu.sync_copy(accum.at[pl.ds(row0, RPT)],
                            out_h.at[pl.ds(row0, RPT), q])


def _sc_agg2(src, dst, tflat, zc):
    return pl.kernel(
        _agg2_body,
        out_type=jax.ShapeDtypeStruct((NP, 16, CH), jnp.float32),
        mesh=_mesh(),
        compiler_params=_sc_params(),
        scratch_types=[
            pltpu.VMEM((B,), jnp.int32),
            pltpu.VMEM((B,), jnp.int32),
            pltpu.VMEM((B,), jnp.int32),
            pltpu.VMEM((B, CH), jnp.float32),
            pltpu.VMEM_SHARED((NP, CH), jnp.float32),
        ],
    )(src, dst, tflat, zc)


_BN = 256


def _tcA_body(dp0, dp1, x8, dinv_o, hs1_o):
    deg = dp0[...] + dp1[...] + 1.0
    dinv = lax.rsqrt(deg)
    dinv_o[...] = dinv
    hs1_o[...] = dinv * x8[...]


def _tcA(dp0, dp1, x8):
    return pl.pallas_call(
        _tcA_body,
        grid=(NP // _BN,),
        in_specs=[
            pl.BlockSpec((_BN, 1), lambda i: (i, 0)),
            pl.BlockSpec((_BN, 1), lambda i: (i, 0)),
            pl.BlockSpec((_BN, CH), lambda i: (i, 0)),
        ],
        out_specs=[
            pl.BlockSpec((_BN, 1), lambda i: (i, 0)),
            pl.BlockSpec((_BN, CH), lambda i: (i, 0)),
        ],
        out_shape=[
            jax.ShapeDtypeStruct((NP, 1), jnp.float32),
            jax.ShapeDtypeStruct((NP, CH), jnp.float32),
        ],
    )(dp0, dp1, x8)


def _tcB_body(u1a, u1b, hs1, dinv, W1, b1, out_o):
    agg = dinv[...] * (u1a[...] + u1b[...] + hs1[...])
    h1 = jax.nn.relu(
        jnp.dot(agg[:, :5], W1[...], preferred_element_type=jnp.float32)
        + b1[...])
    hs2 = dinv[...] * h1
    i = pl.program_id(0)
    rows = i * _BN + lax.broadcasted_iota(jnp.int32, (_BN, 1), 0)
    hs2 = jnp.where(rows < N, hs2, 0.0)
    out_o[...] = jnp.concatenate(
        [hs2, jnp.zeros((_BN, 64), jnp.float32)], axis=1)


def _tcB(u1a, u1b, hs1, dinv, W1, b1):
    spec8 = pl.BlockSpec((_BN, CH), lambda i: (i, 0))
    return pl.pallas_call(
        _tcB_body,
        grid=(NP // _BN,),
        in_specs=[
            spec8, spec8, spec8,
            pl.BlockSpec((_BN, 1), lambda i: (i, 0)),
            pl.BlockSpec((5, 64), lambda i: (0, 0)),
            pl.BlockSpec((64,), lambda i: (0,)),
        ],
        out_specs=pl.BlockSpec((_BN, 128), lambda i: (i, 0)),
        out_shape=jax.ShapeDtypeStruct((NP, 128), jnp.float32),
    )(u1a, u1b, hs1, dinv, W1, b1)


_BNC = 400


def _tcC_body(u2p, hs2p, dinv, gf, W2, b2, Wf1, bf1, Wf2, bf2, Wo, bo,
              out_o):
    d = dinv[...]
    agg = d * (u2p[:, :64] + hs2p[:, :64])
    h2v = jax.nn.relu(
        jnp.dot(agg, W2[...], preferred_element_type=jnp.float32) + b2[...])
    t = jax.nn.relu(
        jnp.dot(h2v, Wf1[:64, :], preferred_element_type=jnp.float32)
        + jnp.dot(gf[...], Wf1[64:67, :], preferred_element_type=jnp.float32)
        + bf1[...])
    t = jax.nn.relu(
        jnp.dot(t, Wf2[...], preferred_element_type=jnp.float32) + bf2[...])
    out_o[...] = (
        jnp.dot(t, Wo[...], preferred_element_type=jnp.float32) + bo[...])


def _tcC(u2p, hs2p, dinv, gf, W2, b2, Wf1, bf1, Wf2, bf2, Wo, bo):
    return pl.pallas_call(
        _tcC_body,
        grid=(N // _BNC,),
        in_specs=[
            pl.BlockSpec((_BNC, 128), lambda i: (i, 0)),
            pl.BlockSpec((_BNC, 128), lambda i: (i, 0)),
            pl.BlockSpec((_BNC, 1), lambda i: (i, 0)),
            pl.BlockSpec((_BNC, 3), lambda i: (i, 0)),
            pl.BlockSpec((64, 64), lambda i: (0, 0)),
            pl.BlockSpec((64,), lambda i: (0,)),
            pl.BlockSpec((67, 64), lambda i: (0, 0)),
            pl.BlockSpec((64,), lambda i: (0,)),
            pl.BlockSpec((64, 64), lambda i: (0, 0)),
            pl.BlockSpec((64,), lambda i: (0,)),
            pl.BlockSpec((64, 30), lambda i: (0, 0)),
            pl.BlockSpec((30,), lambda i: (0,)),
        ],
        out_specs=pl.BlockSpec((_BNC, 30), lambda i: (i, 0)),
        out_shape=jax.ShapeDtypeStruct((N, 30), jnp.float32),
    )(u2p, hs2p, dinv, gf, W2, b2, Wf1, bf1, Wf2, bf2, Wo, bo)


def kernel(x, edge_index, globf, W1, b1, W2, b2, Wf1, bf1, Wf2, bf2, Wo, bo):
    ei = edge_index.astype(jnp.int32)
    pad_tgt = N + (jnp.arange(EP - E, dtype=jnp.int32) % NPAD)
    src = jnp.concatenate([ei[0], pad_tgt])
    dst = jnp.concatenate([ei[1], pad_tgt])
    z1 = jnp.zeros((NP,), jnp.float32)
    zc = jnp.zeros((NP, CH), jnp.float32)

    dg0, dg1 = _sc_degree(dst, z1)
    x8 = jnp.pad(x, ((0, NPAD), (0, CH - x.shape[1])))
    dinv, hs1 = _tcA(dg0.reshape(NP, 1), dg1.reshape(NP, 1), x8)

    u1a, u1b = _sc_agg1(src, dst, hs1, zc)
    hs2p = _tcB(u1a, u1b, hs1, dinv, W1, b1)

    tflat = hs2p.reshape(16 * NP, CH)
    u2 = _sc_agg2(src, dst, tflat, zc)
    u2p = u2.reshape(NP, 128)
    gf = jnp.pad(globf, ((0, NPAD), (0, 0)))
    return _tcC(u2p, hs2p, dinv, gf, W2, b2, Wf1, bf1, Wf2, bf2, Wo, bo)


# speedup vs baseline: 1.0198x; 1.0198x over previous
import jax
import jax.numpy as jnp
from jax import lax
from jax.experimental import pallas as pl
from jax.experimental.pallas import tpu as pltpu
from jax.experimental.pallas import tpu_sc as plsc

N = 100000
E = 1600000
NP = 100096
NPAD = NP - N
EP = 1638400
B = 2048
NSC = 2
NT = 16
RPT = NP // NT
DROW = 6272
DLAST = NP - 15 * DROW
CH = 8
NCH = 64 // CH

_mesh = lambda: plsc.VectorSubcoreMesh(core_axis_name="c", subcore_axis_name="s")
_sc_params = lambda: pltpu.CompilerParams(use_tc_tiling_on_sc=False)


def _fill1d(ref, n16, value):
    def body(i, _):
        ref[pl.ds(i * 16, 16)] = jnp.full((16,), value, jnp.float32)
        return 0
    lax.fori_loop(0, n16, body, 0)


def _deg_body(dst_h, z1_h, out0_h, out1_h, dstb_v, ones_v, accum):
    c = lax.axis_index("c")
    s = lax.axis_index("s")
    row0 = s * DROW
    _fill1d(ones_v, B // 16, 1.0)

    @pl.when(s < 15)
    def _():
        pltpu.sync_copy(z1_h.at[pl.ds(row0, DROW)], accum.at[pl.ds(row0, DROW)])

    @pl.when(s == 15)
    def _():
        pltpu.sync_copy(z1_h.at[pl.ds(row0, DLAST)], accum.at[pl.ds(row0, DLAST)])

    plsc.subcore_barrier()
    e0 = (c * NT + s) * (EP // (NSC * NT))
    nb = EP // (NSC * NT) // B

    def body(j, _):
        base = pl.multiple_of(e0 + j * B, 128)
        pltpu.sync_copy(dst_h.at[pl.ds(base, B)], dstb_v)
        pltpu.sync_copy(ones_v, accum.at[dstb_v], add=True)
        return 0

    lax.fori_loop(0, nb, body, 0)
    plsc.subcore_barrier()
    for cc, out_h in ((0, out0_h), (1, out1_h)):
        @pl.when(c == cc)
        def _(out_h=out_h):
            @pl.when(s < 15)
            def _():
                pltpu.sync_copy(accum.at[pl.ds(row0, DROW)],
                                out_h.at[pl.ds(row0, DROW)])

            @pl.when(s == 15)
            def _():
                pltpu.sync_copy(accum.at[pl.ds(row0, DLAST)],
                                out_h.at[pl.ds(row0, DLAST)])


def _sc_degree(dst, z1):
    return pl.kernel(
        _deg_body,
        out_type=(jax.ShapeDtypeStruct((NP,), jnp.float32),
                  jax.ShapeDtypeStruct((NP,), jnp.float32)),
        mesh=_mesh(),
        compiler_params=_sc_params(),
        scratch_types=[
            pltpu.VMEM((B,), jnp.int32),
            pltpu.VMEM((B,), jnp.float32),
            pltpu.VMEM_SHARED((NP,), jnp.float32),
        ],
    )(dst, z1)


def _agg_pass(src_h, dst_h, table_h, zc_h, out_h, idx_v, dstb_v, rows_v,
              accum, e0, n_edges):
    s = lax.axis_index("s")
    row0 = s * RPT
    pltpu.sync_copy(zc_h.at[pl.ds(row0, RPT)], accum.at[pl.ds(row0, RPT)])
    plsc.subcore_barrier()

    def body(j, _):
        base = pl.multiple_of(e0 + j * B, 128)
        pltpu.sync_copy(src_h.at[pl.ds(base, B)], idx_v)
        pltpu.sync_copy(dst_h.at[pl.ds(base, B)], dstb_v)
        pltpu.sync_copy(table_h.at[idx_v], rows_v)
        pltpu.sync_copy(rows_v, accum.at[dstb_v], add=True)
        return 0

    lax.fori_loop(0, n_edges // B, body, 0)
    plsc.subcore_barrier()
    pltpu.sync_copy(accum.at[pl.ds(row0, RPT)], out_h.at[pl.ds(row0, RPT)])


def _agg1_body(src_h, dst_h, t_h, zc_h, out0_h, out1_h, idx_v, dstb_v,
               rows_v, accum):
    c = lax.axis_index("c")
    s = lax.axis_index("s")
    ept = EP // (NSC * NT)
    e0 = (c * NT + s) * ept
    for cc, out_h in ((0, out0_h), (1, out1_h)):
        @pl.when(c == cc)
        def _(out_h=out_h):
            _agg_pass(src_h, dst_h, t_h, zc_h, out_h, idx_v, dstb_v,
                      rows_v, accum, e0, ept)


def _sc_agg1(src, dst, table, zc):
    return pl.kernel(
        _agg1_body,
        out_type=(jax.ShapeDtypeStruct((NP, CH), jnp.float32),
                  jax.ShapeDtypeStruct((NP, CH), jnp.float32)),
        mesh=_mesh(),
        compiler_params=_sc_params(),
        scratch_types=[
            pltpu.VMEM((B,), jnp.int32),
            pltpu.VMEM((B,), jnp.int32),
            pltpu.VMEM((B, CH), jnp.float32),
            pltpu.VMEM_SHARED((NP, CH), jnp.float32),
        ],
    )(src, dst, table, zc)


def _agg2_body(src_h, dst_h, tflat_h, zc_h, out_h, srcb, dstb, idxq, rows,
               accum, sem_s):
    c = lax.axis_index("c")
    s = lax.axis_index("s")
    ept = EP // NT
    e0 = s * ept
    row0 = s * RPT
    nb = ept // B
    for q in range(NCH):
        @pl.when(q // (NCH // NSC) == c)
        def _(q=q):
            pltpu.sync_copy(zc_h.at[pl.ds(row0, RPT)],
                            accum.at[pl.ds(row0, RPT)])
            plsc.subcore_barrier()

            def body(jj, _):
                p = jj % 2

                @pl.when(jj >= 1)
                def _():
                    pltpu.async_copy(rows.at[1 - p],
                                     accum.at[dstb.at[1 - p]],
                                     sem_s, add=True)

                @pl.when(jj < nb)
                def _():
                    base = pl.multiple_of(e0 + jj * B, 128)
                    pltpu.sync_copy(src_h.at[pl.ds(base, B)], srcb.at[p])
                    pltpu.sync_copy(dst_h.at[pl.ds(base, B)], dstb.at[p])

                    def addq(i, _):
                        idxq[p, pl.ds(i * 16, 16)] = (
                            srcb[p, pl.ds(i * 16, 16)] * 16 + q)
                        return 0

                    lax.fori_loop(0, B // 16, addq, 0)
                    pltpu.sync_copy(tflat_h.at[idxq.at[p]], rows.at[p])

                @pl.when(jj >= 1)
                def _():
                    pltpu.make_async_copy(rows.at[1 - p],
                                          accum.at[dstb.at[1 - p]],
                                          sem_s).wait()
                return 0

            lax.fori_loop(0, nb + 1, body, 0)
            plsc.subcore_barrier()
            pltpu.sync_copy(accum.at[pl.ds(row0, RPT)],
                            out_h.at[pl.ds(row0, RPT), q])


def _sc_agg2(src, dst, tflat, zc):
    return pl.kernel(
        _agg2_body,
        out_type=jax.ShapeDtypeStruct((NP, 16, CH), jnp.float32),
        mesh=_mesh(),
        compiler_params=_sc_params(),
        scratch_types=[
            pltpu.VMEM((2, B), jnp.int32),
            pltpu.VMEM((2, B), jnp.int32),
            pltpu.VMEM((2, B), jnp.int32),
            pltpu.VMEM((2, B, CH), jnp.float32),
            pltpu.VMEM_SHARED((NP, CH), jnp.float32),
            pltpu.SemaphoreType.DMA,
        ],
    )(src, dst, tflat, zc)


_BN = 256


def _tcA_body(dp0, dp1, x8, dinv_o, hs1_o):
    deg = dp0[...] + dp1[...] + 1.0
    dinv = lax.rsqrt(deg)
    dinv_o[...] = dinv
    hs1_o[...] = dinv * x8[...]


def _tcA(dp0, dp1, x8):
    return pl.pallas_call(
        _tcA_body,
        grid=(NP // _BN,),
        in_specs=[
            pl.BlockSpec((_BN, 1), lambda i: (i, 0)),
            pl.BlockSpec((_BN, 1), lambda i: (i, 0)),
            pl.BlockSpec((_BN, CH), lambda i: (i, 0)),
        ],
        out_specs=[
            pl.BlockSpec((_BN, 1), lambda i: (i, 0)),
            pl.BlockSpec((_BN, CH), lambda i: (i, 0)),
        ],
        out_shape=[
            jax.ShapeDtypeStruct((NP, 1), jnp.float32),
            jax.ShapeDtypeStruct((NP, CH), jnp.float32),
        ],
    )(dp0, dp1, x8)


def _tcB_body(u1a, u1b, hs1, dinv, W1, b1, out_o):
    agg = dinv[...] * (u1a[...] + u1b[...] + hs1[...])
    h1 = jax.nn.relu(
        jnp.dot(agg[:, :5], W1[...], preferred_element_type=jnp.float32)
        + b1[...])
    hs2 = dinv[...] * h1
    i = pl.program_id(0)
    rows = i * _BN + lax.broadcasted_iota(jnp.int32, (_BN, 1), 0)
    hs2 = jnp.where(rows < N, hs2, 0.0)
    out_o[...] = jnp.concatenate(
        [hs2, jnp.zeros((_BN, 64), jnp.float32)], axis=1)


def _tcB(u1a, u1b, hs1, dinv, W1, b1):
    spec8 = pl.BlockSpec((_BN, CH), lambda i: (i, 0))
    return pl.pallas_call(
        _tcB_body,
        grid=(NP // _BN,),
        in_specs=[
            spec8, spec8, spec8,
            pl.BlockSpec((_BN, 1), lambda i: (i, 0)),
            pl.BlockSpec((5, 64), lambda i: (0, 0)),
            pl.BlockSpec((64,), lambda i: (0,)),
        ],
        out_specs=pl.BlockSpec((_BN, 128), lambda i: (i, 0)),
        out_shape=jax.ShapeDtypeStruct((NP, 128), jnp.float32),
    )(u1a, u1b, hs1, dinv, W1, b1)


_BNC = 400


def _tcC_body(u2p, hs2p, dinv, gf, W2, b2, Wf1, bf1, Wf2, bf2, Wo, bo,
              out_o):
    d = dinv[...]
    agg = d * (u2p[:, :64] + hs2p[:, :64])
    h2v = jax.nn.relu(
        jnp.dot(agg, W2[...], preferred_element_type=jnp.float32) + b2[...])
    t = jax.nn.relu(
        jnp.dot(h2v, Wf1[:64, :], preferred_element_type=jnp.float32)
        + jnp.dot(gf[...], Wf1[64:67, :], preferred_element_type=jnp.float32)
        + bf1[...])
    t = jax.nn.relu(
        jnp.dot(t, Wf2[...], preferred_element_type=jnp.float32) + bf2[...])
    out_o[...] = (
        jnp.dot(t, Wo[...], preferred_element_type=jnp.float32) + bo[...])


def _tcC(u2p, hs2p, dinv, gf, W2, b2, Wf1, bf1, Wf2, bf2, Wo, bo):
    return pl.pallas_call(
        _tcC_body,
        grid=(N // _BNC,),
        in_specs=[
            pl.BlockSpec((_BNC, 128), lambda i: (i, 0)),
            pl.BlockSpec((_BNC, 128), lambda i: (i, 0)),
            pl.BlockSpec((_BNC, 1), lambda i: (i, 0)),
            pl.BlockSpec((_BNC, 3), lambda i: (i, 0)),
            pl.BlockSpec((64, 64), lambda i: (0, 0)),
            pl.BlockSpec((64,), lambda i: (0,)),
            pl.BlockSpec((67, 64), lambda i: (0, 0)),
            pl.BlockSpec((64,), lambda i: (0,)),
            pl.BlockSpec((64, 64), lambda i: (0, 0)),
            pl.BlockSpec((64,), lambda i: (0,)),
            pl.BlockSpec((64, 30), lambda i: (0, 0)),
            pl.BlockSpec((30,), lambda i: (0,)),
        ],
        out_specs=pl.BlockSpec((_BNC, 30), lambda i: (i, 0)),
        out_shape=jax.ShapeDtypeStruct((N, 30), jnp.float32),
    )(u2p, hs2p, dinv, gf, W2, b2, Wf1, bf1, Wf2, bf2, Wo, bo)


def kernel(x, edge_index, globf, W1, b1, W2, b2, Wf1, bf1, Wf2, bf2, Wo, bo):
    ei = edge_index.astype(jnp.int32)
    pad_tgt = N + (jnp.arange(EP - E, dtype=jnp.int32) % NPAD)
    src = jnp.concatenate([ei[0], pad_tgt])
    dst = jnp.concatenate([ei[1], pad_tgt])
    z1 = jnp.zeros((NP,), jnp.float32)
    zc = jnp.zeros((NP, CH), jnp.float32)

    dg0, dg1 = _sc_degree(dst, z1)
    x8 = jnp.pad(x, ((0, NPAD), (0, CH - x.shape[1])))
    dinv, hs1 = _tcA(dg0.reshape(NP, 1), dg1.reshape(NP, 1), x8)

    u1a, u1b = _sc_agg1(src, dst, hs1, zc)
    hs2p = _tcB(u1a, u1b, hs1, dinv, W1, b1)

    tflat = hs2p.reshape(16 * NP, CH)
    u2 = _sc_agg2(src, dst, tflat, zc)
    u2p = u2.reshape(NP, 128)
    gf = jnp.pad(globf, ((0, NPAD), (0, 0)))
    return _tcC(u2p, hs2p, dinv, gf, W2, b2, Wf1, bf1, Wf2, bf2, Wo, bo)


# speedup vs baseline: 1.5769x; 1.5463x over previous
import jax
import jax.numpy as jnp
from jax import lax
from jax.experimental import pallas as pl
from jax.experimental.pallas import tpu as pltpu
from jax.experimental.pallas import tpu_sc as plsc

N = 100000
E = 1600000
NP = 100096
NPAD = NP - N
EP = 1638400
BD = 2048
B4 = 512
NSC = 2
NT = 16
RPT = NP // NT
DROW = 6272
DLAST = NP - 15 * DROW
TL = 8 * NP - 7

_mesh = lambda: plsc.VectorSubcoreMesh(core_axis_name="c", subcore_axis_name="s")
_sc_params = lambda: pltpu.CompilerParams(use_tc_tiling_on_sc=False)


def _fill1d(ref, n16, value):
    def body(i, _):
        ref[pl.ds(i * 16, 16)] = jnp.full((16,), value, jnp.float32)
        return 0
    lax.fori_loop(0, n16, body, 0)


def _deg_body(dst_h, z1_h, out0_h, out1_h, dstb_v, ones_v, accum):
    c = lax.axis_index("c")
    s = lax.axis_index("s")
    row0 = s * DROW
    _fill1d(ones_v, BD // 16, 1.0)

    @pl.when(s < 15)
    def _():
        pltpu.sync_copy(z1_h.at[pl.ds(row0, DROW)], accum.at[pl.ds(row0, DROW)])

    @pl.when(s == 15)
    def _():
        pltpu.sync_copy(z1_h.at[pl.ds(row0, DLAST)], accum.at[pl.ds(row0, DLAST)])

    plsc.subcore_barrier()
    e0 = (c * NT + s) * (EP // (NSC * NT))
    nb = EP // (NSC * NT) // BD

    def body(j, _):
        base = pl.multiple_of(e0 + j * BD, 128)
        pltpu.sync_copy(dst_h.at[pl.ds(base, BD)], dstb_v)
        pltpu.sync_copy(ones_v, accum.at[dstb_v], add=True)
        return 0

    lax.fori_loop(0, nb, body, 0)
    plsc.subcore_barrier()
    for cc, out_h in ((0, out0_h), (1, out1_h)):
        @pl.when(c == cc)
        def _(out_h=out_h):
            @pl.when(s < 15)
            def _():
                pltpu.sync_copy(accum.at[pl.ds(row0, DROW)],
                                out_h.at[pl.ds(row0, DROW)])

            @pl.when(s == 15)
            def _():
                pltpu.sync_copy(accum.at[pl.ds(row0, DLAST)],
                                out_h.at[pl.ds(row0, DLAST)])


def _sc_degree(dst, z1):
    return pl.kernel(
        _deg_body,
        out_type=(jax.ShapeDtypeStruct((NP,), jnp.float32),
                  jax.ShapeDtypeStruct((NP,), jnp.float32)),
        mesh=_mesh(),
        compiler_params=_sc_params(),
        scratch_types=[
            pltpu.VMEM((BD,), jnp.int32),
            pltpu.VMEM((BD,), jnp.float32),
            pltpu.VMEM_SHARED((NP,), jnp.float32),
        ],
    )(dst, z1)


def _agg_pass(ed_h, tview, out_h, out_col, srcb, dstb, rows, accum, sem_s,
              e0, nb):
    s = lax.axis_index("s")
    row0 = s * RPT

    def zf(i, _):
        rows[1, i] = jnp.zeros((16,), jnp.float32)
        dstb[1, pl.ds((i % 64) * 16, 16)] = jnp.zeros((16,), jnp.int32)
        return 0

    lax.fori_loop(0, B4, zf, 0)
    _NF = RPT // B4
    _TAIL = RPT - _NF * B4
    for r in range(_NF):
        pltpu.sync_copy(rows.at[1], accum.at[pl.ds(row0 + r * B4, B4)])
    if _TAIL:
        pltpu.sync_copy(rows.at[1].at[pl.ds(0, _TAIL)],
                        accum.at[pl.ds(row0 + _NF * B4, _TAIL)])
    plsc.subcore_barrier()

    def body(jj, _):
        p = jj % 2
        jc = jnp.minimum(jj, nb - 1)
        base = pl.multiple_of(e0 + jc * B4, 128)
        pltpu.sync_copy(ed_h.at[0, pl.ds(base, B4)], srcb.at[p])
        pltpu.sync_copy(ed_h.at[1, pl.ds(base, B4)], dstb.at[p])
        gd = pltpu.async_copy(tview.at[srcb.at[p]], rows.at[p], sem_s)
        pltpu.sync_copy(rows.at[1 - p], accum.at[dstb.at[1 - p]], add=True)
        gd.wait()
        return 0

    lax.fori_loop(0, nb + 1, body, 0)
    plsc.subcore_barrier()
    _NF = RPT // B4
    _TAIL = RPT - _NF * B4
    for r in range(_NF):
        pltpu.sync_copy(accum.at[pl.ds(row0 + r * B4, B4)], rows.at[0])
        pltpu.sync_copy(rows.at[0],
                        out_h.at[pl.ds(row0 + r * B4, B4),
                                 pl.ds(out_col, 16)])
    if _TAIL:
        pltpu.sync_copy(accum.at[pl.ds(row0 + _NF * B4, _TAIL)],
                        rows.at[0].at[pl.ds(0, _TAIL)])
        pltpu.sync_copy(rows.at[0].at[pl.ds(0, _TAIL)],
                        out_h.at[pl.ds(row0 + _NF * B4, _TAIL),
                                 pl.ds(out_col, 16)])


def _agg1_body(ed_h, tflat_h, out_h, srcb, dstb, rows, accum, sem_s):
    c = lax.axis_index("c")
    s = lax.axis_index("s")
    ept = EP // (NSC * NT)
    _agg_pass(ed_h, tflat_h.at[pl.ds(0, TL)], out_h, 16 * c, srcb, dstb,
              rows, accum, sem_s, (c * NT + s) * ept, ept // B4)


def _agg2_body(ed_h, tflat_h, out_h, srcb, dstb, rows, accum, sem_s):
    c = lax.axis_index("c")
    s = lax.axis_index("s")
    ept = EP // NT
    e0 = s * ept

    def chunk(k, _):
        p = 2 * c + k
        _agg_pass(ed_h, tflat_h.at[pl.ds(p, TL)], out_h, 16 * p,
                  srcb, dstb, rows, accum, sem_s, e0, ept // B4)
        return 0

    lax.fori_loop(0, 2, chunk, 0)


def _sc_agg(body, ed, tflat):
    return pl.kernel(
        body,
        out_type=jax.ShapeDtypeStruct((NP, 128), jnp.float32),
        mesh=_mesh(),
        compiler_params=_sc_params(),
        scratch_types=[
            pltpu.VMEM((2, B4), jnp.int32),
            pltpu.VMEM((2, B4), jnp.int32),
            pltpu.VMEM((2, B4, 16), jnp.float32),
            pltpu.VMEM_SHARED((NP, 16), jnp.float32),
            pltpu.SemaphoreType.DMA,
        ],
    )(ed, tflat)


_BN = 256
_BNC = 400


def _tcB_body(u1p, hs1p, gf, W1, b1, out_o):
    dinv = hs1p[:, 8:9]
    agg5 = dinv * (u1p[:, 0:5] + u1p[:, 16:21] + hs1p[:, 0:5])
    h1 = jax.nn.relu(
        jnp.dot(agg5, W1[...], preferred_element_type=jnp.float32) + b1[...])
    hs2 = dinv * h1
    i = pl.program_id(0)
    rows = i * _BN + lax.broadcasted_iota(jnp.int32, (_BN, 1), 0)
    hs2 = jnp.where(rows < N, hs2, 0.0)
    out_o[...] = jnp.concatenate(
        [hs2, dinv, gf[...], jnp.zeros((_BN, 60), jnp.float32)], axis=1)


def _tcB(u1p, hs1p, gfp, W1, b1):
    return pl.pallas_call(
        _tcB_body,
        grid=(NP // _BN,),
        in_specs=[
            pl.BlockSpec((_BN, 128), lambda i: (i, 0)),
            pl.BlockSpec((_BN, 128), lambda i: (i, 0)),
            pl.BlockSpec((_BN, 3), lambda i: (i, 0)),
            pl.BlockSpec((5, 64), lambda i: (0, 0)),
            pl.BlockSpec((64,), lambda i: (0,)),
        ],
        out_specs=pl.BlockSpec((_BN, 128), lambda i: (i, 0)),
        out_shape=jax.ShapeDtypeStruct((NP, 128), jnp.float32),
    )(u1p, hs1p, gfp, W1, b1)


def _tcC_body(u2p, hs2p, W2, b2, Wf1, bf1, Wf2, bf2, Wo, bo, out_o):
    dinv = hs2p[:, 64:65]
    gf = hs2p[:, 65:68]
    agg = dinv * (u2p[:, :64] + hs2p[:, :64])
    h2v = jax.nn.relu(
        jnp.dot(agg, W2[...], preferred_element_type=jnp.float32) + b2[...])
    t = jax.nn.relu(
        jnp.dot(h2v, Wf1[:64, :], preferred_element_type=jnp.float32)
        + jnp.dot(gf, Wf1[64:67, :], preferred_element_type=jnp.float32)
        + bf1[...])
    t = jax.nn.relu(
        jnp.dot(t, Wf2[...], preferred_element_type=jnp.float32) + bf2[...])
    out_o[...] = (
        jnp.dot(t, Wo[...], preferred_element_type=jnp.float32) + bo[...])


def _tcC(u2p, hs2p, W2, b2, Wf1, bf1, Wf2, bf2, Wo, bo):
    return pl.pallas_call(
        _tcC_body,
        grid=(N // _BNC,),
        in_specs=[
            pl.BlockSpec((_BNC, 128), lambda i: (i, 0)),
            pl.BlockSpec((_BNC, 128), lambda i: (i, 0)),
            pl.BlockSpec((64, 64), lambda i: (0, 0)),
            pl.BlockSpec((64,), lambda i: (0,)),
            pl.BlockSpec((67, 64), lambda i: (0, 0)),
            pl.BlockSpec((64,), lambda i: (0,)),
            pl.BlockSpec((64, 64), lambda i: (0, 0)),
            pl.BlockSpec((64,), lambda i: (0,)),
            pl.BlockSpec((64, 30), lambda i: (0, 0)),
            pl.BlockSpec((30,), lambda i: (0,)),
        ],
        out_specs=pl.BlockSpec((_BNC, 30), lambda i: (i, 0)),
        out_shape=jax.ShapeDtypeStruct((N, 30), jnp.float32),
    )(u2p, hs2p, W2, b2, Wf1, bf1, Wf2, bf2, Wo, bo)


def kernel(x, edge_index, globf, W1, b1, W2, b2, Wf1, bf1, Wf2, bf2, Wo, bo):
    ei = edge_index.astype(jnp.int32)
    pad_tgt = N + (jnp.arange(EP - E, dtype=jnp.int32) % NPAD)
    src = jnp.concatenate([ei[0], pad_tgt])
    dst = jnp.concatenate([ei[1], pad_tgt])
    ed = jnp.stack([src * 8, dst])
    z1 = jnp.zeros((NP,), jnp.float32)

    dg0, dg1 = _sc_degree(dst, z1)
    dinv = lax.rsqrt(dg0 + dg1 + 1.0)
    x5 = jnp.pad(x, ((0, NPAD), (0, 0)))
    hs1p = jnp.concatenate(
        [dinv[:, None] * x5, jnp.zeros((NP, 3), jnp.float32),
         dinv[:, None], jnp.zeros((NP, 119), jnp.float32)], axis=1)

    u1p = _sc_agg(_agg1_body, ed, hs1p.reshape(8 * NP, 16))
    gfp = jnp.pad(globf, ((0, NPAD), (0, 0)))
    hs2p = _tcB(u1p, hs1p, gfp, W1, b1)

    u2p = _sc_agg(_agg2_body, ed, hs2p.reshape(8 * NP, 16))
    return _tcC(u2p, hs2p, W2, b2, Wf1, bf1, Wf2, bf2, Wo, bo)


# speedup vs baseline: 1.9511x; 1.2373x over previous
import jax
import jax.numpy as jnp
from jax import lax
from jax.experimental import pallas as pl
from jax.experimental.pallas import tpu as pltpu
from jax.experimental.pallas import tpu_sc as plsc

N = 100000
E = 1600000
NP = 100096
NPAD = NP - N
EP = 1638400
BD = 2048
B4 = 640
NSC = 2
NT = 16
RPT = NP // NT
DROW = 6272
DLAST = NP - 15 * DROW
TL = 8 * NP - 7

_mesh = lambda: plsc.VectorSubcoreMesh(core_axis_name="c", subcore_axis_name="s")
_sc_params = lambda: pltpu.CompilerParams(use_tc_tiling_on_sc=False)


def _fill1d(ref, n16, value):
    def body(i, _):
        ref[pl.ds(i * 16, 16)] = jnp.full((16,), value, jnp.float32)
        return 0
    lax.fori_loop(0, n16, body, 0)


def _deg_body(dst_h, z1_h, out0_h, out1_h, dstb_v, ones_v, accum):
    c = lax.axis_index("c")
    s = lax.axis_index("s")
    row0 = s * DROW
    _fill1d(ones_v, BD // 16, 1.0)

    @pl.when(s < 15)
    def _():
        pltpu.sync_copy(z1_h.at[pl.ds(row0, DROW)], accum.at[pl.ds(row0, DROW)])

    @pl.when(s == 15)
    def _():
        pltpu.sync_copy(z1_h.at[pl.ds(row0, DLAST)], accum.at[pl.ds(row0, DLAST)])

    plsc.subcore_barrier()
    e0 = (c * NT + s) * (EP // (NSC * NT))
    nb = EP // (NSC * NT) // BD

    def body(j, _):
        base = pl.multiple_of(e0 + j * BD, 128)
        pltpu.sync_copy(dst_h.at[pl.ds(base, BD)], dstb_v)
        pltpu.sync_copy(ones_v, accum.at[dstb_v], add=True)
        return 0

    lax.fori_loop(0, nb, body, 0)
    plsc.subcore_barrier()
    for cc, out_h in ((0, out0_h), (1, out1_h)):
        @pl.when(c == cc)
        def _(out_h=out_h):
            @pl.when(s < 15)
            def _():
                pltpu.sync_copy(accum.at[pl.ds(row0, DROW)],
                                out_h.at[pl.ds(row0, DROW)])

            @pl.when(s == 15)
            def _():
                pltpu.sync_copy(accum.at[pl.ds(row0, DLAST)],
                                out_h.at[pl.ds(row0, DLAST)])


def _sc_degree(dst, z1):
    return pl.kernel(
        _deg_body,
        out_type=(jax.ShapeDtypeStruct((NP,), jnp.float32),
                  jax.ShapeDtypeStruct((NP,), jnp.float32)),
        mesh=_mesh(),
        compiler_params=_sc_params(),
        scratch_types=[
            pltpu.VMEM((BD,), jnp.int32),
            pltpu.VMEM((BD,), jnp.float32),
            pltpu.VMEM_SHARED((NP,), jnp.float32),
        ],
    )(dst, z1)


def _agg_pass(ed_h, tview, out_h, out_col, srcb, dstb, rows, accum, sem_s,
              e0, nb):
    s = lax.axis_index("s")
    row0 = s * RPT

    def zf(i, _):
        rows[1, i] = jnp.zeros((16,), jnp.float32)
        dstb[1, pl.ds((i % 64) * 16, 16)] = jnp.zeros((16,), jnp.int32)
        return 0

    lax.fori_loop(0, B4, zf, 0)
    _NF = RPT // B4
    _TAIL = RPT - _NF * B4
    for r in range(_NF):
        pltpu.sync_copy(rows.at[1], accum.at[pl.ds(row0 + r * B4, B4)])
    if _TAIL:
        pltpu.sync_copy(rows.at[1].at[pl.ds(0, _TAIL)],
                        accum.at[pl.ds(row0 + _NF * B4, _TAIL)])
    plsc.subcore_barrier()

    def body(jj, _):
        p = jj % 2
        jc = jnp.minimum(jj, nb - 1)
        base = pl.multiple_of(e0 + jc * B4, 128)
        pltpu.sync_copy(ed_h.at[0, pl.ds(base, B4)], srcb.at[p])
        pltpu.sync_copy(ed_h.at[1, pl.ds(base, B4)], dstb.at[p])
        gd = pltpu.async_copy(tview.at[srcb.at[p]], rows.at[p], sem_s)
        pltpu.sync_copy(rows.at[1 - p], accum.at[dstb.at[1 - p]], add=True)
        gd.wait()
        return 0

    lax.fori_loop(0, nb + 1, body, 0)
    plsc.subcore_barrier()
    _NF = RPT // B4
    _TAIL = RPT - _NF * B4
    for r in range(_NF):
        pltpu.sync_copy(accum.at[pl.ds(row0 + r * B4, B4)], rows.at[0])
        pltpu.sync_copy(rows.at[0],
                        out_h.at[pl.ds(row0 + r * B4, B4),
                                 pl.ds(out_col, 16)])
    if _TAIL:
        pltpu.sync_copy(accum.at[pl.ds(row0 + _NF * B4, _TAIL)],
                        rows.at[0].at[pl.ds(0, _TAIL)])
        pltpu.sync_copy(rows.at[0].at[pl.ds(0, _TAIL)],
                        out_h.at[pl.ds(row0 + _NF * B4, _TAIL),
                                 pl.ds(out_col, 16)])


def _agg1_body(ed_h, tflat_h, out_h, srcb, dstb, rows, accum, sem_s):
    c = lax.axis_index("c")
    s = lax.axis_index("s")
    ept = EP // (NSC * NT)
    _agg_pass(ed_h, tflat_h.at[pl.ds(0, TL)], out_h, 16 * c, srcb, dstb,
              rows, accum, sem_s, (c * NT + s) * ept, ept // B4)


def _agg2_body(ed_h, tflat_h, out_h, srcb, dstb, rows, accum, sem_s):
    c = lax.axis_index("c")
    s = lax.axis_index("s")
    ept = EP // NT
    e0 = s * ept

    def chunk(k, _):
        p = 2 * c + k
        _agg_pass(ed_h, tflat_h.at[pl.ds(p, TL)], out_h, 16 * p,
                  srcb, dstb, rows, accum, sem_s, e0, ept // B4)
        return 0

    lax.fori_loop(0, 2, chunk, 0)


def _sc_agg(body, ed, tflat):
    return pl.kernel(
        body,
        out_type=jax.ShapeDtypeStruct((NP, 128), jnp.float32),
        mesh=_mesh(),
        compiler_params=_sc_params(),
        scratch_types=[
            pltpu.VMEM((2, B4), jnp.int32),
            pltpu.VMEM((2, B4), jnp.int32),
            pltpu.VMEM((2, B4, 16), jnp.float32),
            pltpu.VMEM_SHARED((NP, 16), jnp.float32),
            pltpu.SemaphoreType.DMA,
        ],
    )(ed, tflat)


_BN = 1088
_BNC = 1000


def _tcB_body(u1p, hs1p, gf, W1, b1, out_o):
    dinv = hs1p[:, 8:9]
    agg5 = dinv * (u1p[:, 0:5] + u1p[:, 16:21] + hs1p[:, 0:5])
    h1 = jax.nn.relu(
        jnp.dot(agg5, W1[...], preferred_element_type=jnp.float32) + b1[...])
    hs2 = dinv * h1
    i = pl.program_id(0)
    rows = i * _BN + lax.broadcasted_iota(jnp.int32, (_BN, 1), 0)
    hs2 = jnp.where(rows < N, hs2, 0.0)
    out_o[...] = jnp.concatenate(
        [hs2, dinv, gf[...], jnp.zeros((_BN, 60), jnp.float32)], axis=1)


def _tcB(u1p, hs1p, gfp, W1, b1):
    return pl.pallas_call(
        _tcB_body,
        grid=(NP // _BN,),
        in_specs=[
            pl.BlockSpec((_BN, 128), lambda i: (i, 0)),
            pl.BlockSpec((_BN, 128), lambda i: (i, 0)),
            pl.BlockSpec((_BN, 3), lambda i: (i, 0)),
            pl.BlockSpec((5, 64), lambda i: (0, 0)),
            pl.BlockSpec((64,), lambda i: (0,)),
        ],
        out_specs=pl.BlockSpec((_BN, 128), lambda i: (i, 0)),
        out_shape=jax.ShapeDtypeStruct((NP, 128), jnp.float32),
    )(u1p, hs1p, gfp, W1, b1)


def _tcC_body(u2p, hs2p, W2, b2, Wf1, bf1, Wf2, bf2, Wo, bo, out_o):
    dinv = hs2p[:, 64:65]
    gf = hs2p[:, 65:68]
    agg = dinv * (u2p[:, :64] + hs2p[:, :64])
    h2v = jax.nn.relu(
        jnp.dot(agg, W2[...], preferred_element_type=jnp.float32) + b2[...])
    t = jax.nn.relu(
        jnp.dot(h2v, Wf1[:64, :], preferred_element_type=jnp.float32)
        + jnp.dot(gf, Wf1[64:67, :], preferred_element_type=jnp.float32)
        + bf1[...])
    t = jax.nn.relu(
        jnp.dot(t, Wf2[...], preferred_element_type=jnp.float32) + bf2[...])
    out_o[...] = (
        jnp.dot(t, Wo[...], preferred_element_type=jnp.float32) + bo[...])


def _tcC(u2p, hs2p, W2, b2, Wf1, bf1, Wf2, bf2, Wo, bo):
    return pl.pallas_call(
        _tcC_body,
        grid=(N // _BNC,),
        in_specs=[
            pl.BlockSpec((_BNC, 128), lambda i: (i, 0)),
            pl.BlockSpec((_BNC, 128), lambda i: (i, 0)),
            pl.BlockSpec((64, 64), lambda i: (0, 0)),
            pl.BlockSpec((64,), lambda i: (0,)),
            pl.BlockSpec((67, 64), lambda i: (0, 0)),
            pl.BlockSpec((64,), lambda i: (0,)),
            pl.BlockSpec((64, 64), lambda i: (0, 0)),
            pl.BlockSpec((64,), lambda i: (0,)),
            pl.BlockSpec((64, 30), lambda i: (0, 0)),
            pl.BlockSpec((30,), lambda i: (0,)),
        ],
        out_specs=pl.BlockSpec((_BNC, 30), lambda i: (i, 0)),
        out_shape=jax.ShapeDtypeStruct((N, 30), jnp.float32),
    )(u2p, hs2p, W2, b2, Wf1, bf1, Wf2, bf2, Wo, bo)


def kernel(x, edge_index, globf, W1, b1, W2, b2, Wf1, bf1, Wf2, bf2, Wo, bo):
    ei = edge_index.astype(jnp.int32)
    pad_tgt = N + (jnp.arange(EP - E, dtype=jnp.int32) % NPAD)
    src = jnp.concatenate([ei[0], pad_tgt])
    dst = jnp.concatenate([ei[1], pad_tgt])
    ed = jnp.stack([src * 8, dst])
    z1 = jnp.zeros((NP,), jnp.float32)

    dg0, dg1 = _sc_degree(dst, z1)
    dinv = lax.rsqrt(dg0 + dg1 + 1.0)
    x5 = jnp.pad(x, ((0, NPAD), (0, 0)))
    hs1p = jnp.concatenate(
        [dinv[:, None] * x5, jnp.zeros((NP, 3), jnp.float32),
         dinv[:, None], jnp.zeros((NP, 119), jnp.float32)], axis=1)

    u1p = _sc_agg(_agg1_body, ed, hs1p.reshape(8 * NP, 16))
    gfp = jnp.pad(globf, ((0, NPAD), (0, 0)))
    hs2p = _tcB(u1p, hs1p, gfp, W1, b1)

    u2p = _sc_agg(_agg2_body, ed, hs2p.reshape(8 * NP, 16))
    return _tcC(u2p, hs2p, W2, b2, Wf1, bf1, Wf2, bf2, Wo, bo)


# speedup vs baseline: 2.0645x; 1.0581x over previous
import jax
import jax.numpy as jnp
from jax import lax
from jax.experimental import pallas as pl
from jax.experimental.pallas import tpu as pltpu
from jax.experimental.pallas import tpu_sc as plsc

N = 100000
E = 1600000
NP = 100096
NPAD = NP - N
EP = 1638400
BD = 2048
B4 = 512
NSC = 2
NT = 16
RPT = NP // NT
DROW = 6272
DLAST = NP - 15 * DROW
TL = 8 * NP - 7

_mesh = lambda: plsc.VectorSubcoreMesh(core_axis_name="c", subcore_axis_name="s")
_sc_params = lambda: pltpu.CompilerParams(use_tc_tiling_on_sc=False)


def _fill1d(ref, n16, value):
    def body(i, _):
        ref[pl.ds(i * 16, 16)] = jnp.full((16,), value, jnp.float32)
        return 0
    lax.fori_loop(0, n16, body, 0)


def _deg_body(dst_h, z1_h, out0_h, out1_h, dstb_v, ones_v, accum):
    c = lax.axis_index("c")
    s = lax.axis_index("s")
    row0 = s * DROW
    _fill1d(ones_v, BD // 16, 1.0)

    @pl.when(s < 15)
    def _():
        pltpu.sync_copy(z1_h.at[pl.ds(row0, DROW)], accum.at[pl.ds(row0, DROW)])

    @pl.when(s == 15)
    def _():
        pltpu.sync_copy(z1_h.at[pl.ds(row0, DLAST)], accum.at[pl.ds(row0, DLAST)])

    plsc.subcore_barrier()
    e0 = (c * NT + s) * (EP // (NSC * NT))
    nb = EP // (NSC * NT) // BD

    def body(j, _):
        base = pl.multiple_of(e0 + j * BD, 128)
        pltpu.sync_copy(dst_h.at[pl.ds(base, BD)], dstb_v)
        pltpu.sync_copy(ones_v, accum.at[dstb_v], add=True)
        return 0

    lax.fori_loop(0, nb, body, 0)
    plsc.subcore_barrier()
    for cc, out_h in ((0, out0_h), (1, out1_h)):
        @pl.when(c == cc)
        def _(out_h=out_h):
            @pl.when(s < 15)
            def _():
                pltpu.sync_copy(accum.at[pl.ds(row0, DROW)],
                                out_h.at[pl.ds(row0, DROW)])

            @pl.when(s == 15)
            def _():
                pltpu.sync_copy(accum.at[pl.ds(row0, DLAST)],
                                out_h.at[pl.ds(row0, DLAST)])


def _sc_degree(dst, z1):
    return pl.kernel(
        _deg_body,
        out_type=(jax.ShapeDtypeStruct((NP,), jnp.float32),
                  jax.ShapeDtypeStruct((NP,), jnp.float32)),
        mesh=_mesh(),
        compiler_params=_sc_params(),
        scratch_types=[
            pltpu.VMEM((BD,), jnp.int32),
            pltpu.VMEM((BD,), jnp.float32),
            pltpu.VMEM_SHARED((NP,), jnp.float32),
        ],
    )(dst, z1)


def _agg_pass(ed_h, tview, out_h, out_col, eb, rows, accum, sem_g, sem_s,
              e0, nb):
    s = lax.axis_index("s")
    row0 = s * RPT

    def zf(i, _):
        rows[2, i] = jnp.zeros((16,), jnp.float32)
        eb[2, 1, pl.ds((i % (B4 // 16)) * 16, 16)] = jnp.zeros((16,),
                                                              jnp.int32)
        return 0

    lax.fori_loop(0, B4, zf, 0)
    _NF = RPT // B4
    _TAIL = RPT - _NF * B4
    for r in range(_NF):
        pltpu.sync_copy(rows.at[2], accum.at[pl.ds(row0 + r * B4, B4)])
    if _TAIL:
        pltpu.sync_copy(rows.at[2].at[pl.ds(0, _TAIL)],
                        accum.at[pl.ds(row0 + _NF * B4, _TAIL)])
    plsc.subcore_barrier()

    def body(jj, _):
        p = jj % 3
        pm = (jj + 2) % 3

        @pl.when(jj >= 2)
        def _():
            pltpu.make_async_copy(rows.at[p], accum.at[eb.at[p, 1]],
                                  sem_s).wait()

        jc = jnp.minimum(jj, nb - 1)
        base = pl.multiple_of(e0 + jc * B4, 128)
        pltpu.sync_copy(ed_h.at[:, pl.ds(base, B4)], eb.at[p])
        gd = pltpu.async_copy(tview.at[eb.at[p, 0]], rows.at[p], sem_g)
        pltpu.async_copy(rows.at[pm], accum.at[eb.at[pm, 1]], sem_s,
                         add=True)
        gd.wait()
        return 0

    lax.fori_loop(0, nb + 1, body, 0)
    for _ in range(2):
        pltpu.make_async_copy(rows.at[0], accum.at[eb.at[0, 1]],
                              sem_s).wait()
    plsc.subcore_barrier()
    _NF = RPT // B4
    _TAIL = RPT - _NF * B4
    for r in range(_NF):
        pltpu.sync_copy(accum.at[pl.ds(row0 + r * B4, B4)], rows.at[0])
        pltpu.sync_copy(rows.at[0],
                        out_h.at[pl.ds(row0 + r * B4, B4),
                                 pl.ds(out_col, 16)])
    if _TAIL:
        pltpu.sync_copy(accum.at[pl.ds(row0 + _NF * B4, _TAIL)],
                        rows.at[0].at[pl.ds(0, _TAIL)])
        pltpu.sync_copy(rows.at[0].at[pl.ds(0, _TAIL)],
                        out_h.at[pl.ds(row0 + _NF * B4, _TAIL),
                                 pl.ds(out_col, 16)])


def _agg1_body(ed_h, tflat_h, out_h, eb, rows, accum, sem_g, sem_s):
    c = lax.axis_index("c")
    s = lax.axis_index("s")
    ept = EP // (NSC * NT)
    _agg_pass(ed_h, tflat_h.at[pl.ds(0, TL)], out_h, 16 * c, eb,
              rows, accum, sem_g, sem_s, (c * NT + s) * ept, ept // B4)


def _agg2_body(ed_h, tflat_h, out_h, eb, rows, accum, sem_g, sem_s):
    c = lax.axis_index("c")
    s = lax.axis_index("s")
    ept = EP // NT
    e0 = s * ept

    def chunk(k, _):
        p = 2 * c + k
        _agg_pass(ed_h, tflat_h.at[pl.ds(p, TL)], out_h, 16 * p,
                  eb, rows, accum, sem_g, sem_s, e0, ept // B4)
        return 0

    lax.fori_loop(0, 2, chunk, 0)


def _sc_agg(body, ed, tflat):
    return pl.kernel(
        body,
        out_type=jax.ShapeDtypeStruct((NP, 128), jnp.float32),
        mesh=_mesh(),
        compiler_params=_sc_params(),
        scratch_types=[
            pltpu.VMEM((3, 2, B4), jnp.int32),
            pltpu.VMEM((3, B4, 16), jnp.float32),
            pltpu.VMEM_SHARED((NP, 16), jnp.float32),
            pltpu.SemaphoreType.DMA,
            pltpu.SemaphoreType.DMA,
        ],
    )(ed, tflat)


_BN = 1088
_BNC = 1000


def _tcB_body(u1p, hs1p, gf, W1, b1, out_o):
    dinv = hs1p[:, 8:9]
    agg5 = dinv * (u1p[:, 0:5] + u1p[:, 16:21] + hs1p[:, 0:5])
    h1 = jax.nn.relu(
        jnp.dot(agg5, W1[...], preferred_element_type=jnp.float32) + b1[...])
    hs2 = dinv * h1
    i = pl.program_id(0)
    rows = i * _BN + lax.broadcasted_iota(jnp.int32, (_BN, 1), 0)
    hs2 = jnp.where(rows < N, hs2, 0.0)
    out_o[...] = jnp.concatenate(
        [hs2, dinv, gf[...], jnp.zeros((_BN, 60), jnp.float32)], axis=1)


def _tcB(u1p, hs1p, gfp, W1, b1):
    return pl.pallas_call(
        _tcB_body,
        grid=(NP // _BN,),
        in_specs=[
            pl.BlockSpec((_BN, 128), lambda i: (i, 0)),
            pl.BlockSpec((_BN, 128), lambda i: (i, 0)),
            pl.BlockSpec((_BN, 3), lambda i: (i, 0)),
            pl.BlockSpec((5, 64), lambda i: (0, 0)),
            pl.BlockSpec((64,), lambda i: (0,)),
        ],
        out_specs=pl.BlockSpec((_BN, 128), lambda i: (i, 0)),
        out_shape=jax.ShapeDtypeStruct((NP, 128), jnp.float32),
    )(u1p, hs1p, gfp, W1, b1)


def _tcC_body(u2p, hs2p, W2, b2, Wf1, bf1, Wf2, bf2, Wo, bo, out_o):
    dinv = hs2p[:, 64:65]
    gf = hs2p[:, 65:68]
    agg = dinv * (u2p[:, :64] + hs2p[:, :64])
    h2v = jax.nn.relu(
        jnp.dot(agg, W2[...], preferred_element_type=jnp.float32) + b2[...])
    t = jax.nn.relu(
        jnp.dot(h2v, Wf1[:64, :], preferred_element_type=jnp.float32)
        + jnp.dot(gf, Wf1[64:67, :], preferred_element_type=jnp.float32)
        + bf1[...])
    t = jax.nn.relu(
        jnp.dot(t, Wf2[...], preferred_element_type=jnp.float32) + bf2[...])
    out_o[...] = (
        jnp.dot(t, Wo[...], preferred_element_type=jnp.float32) + bo[...])


def _tcC(u2p, hs2p, W2, b2, Wf1, bf1, Wf2, bf2, Wo, bo):
    return pl.pallas_call(
        _tcC_body,
        grid=(N // _BNC,),
        in_specs=[
            pl.BlockSpec((_BNC, 128), lambda i: (i, 0)),
            pl.BlockSpec((_BNC, 128), lambda i: (i, 0)),
            pl.BlockSpec((64, 64), lambda i: (0, 0)),
            pl.BlockSpec((64,), lambda i: (0,)),
            pl.BlockSpec((67, 64), lambda i: (0, 0)),
            pl.BlockSpec((64,), lambda i: (0,)),
            pl.BlockSpec((64, 64), lambda i: (0, 0)),
            pl.BlockSpec((64,), lambda i: (0,)),
            pl.BlockSpec((64, 30), lambda i: (0, 0)),
            pl.BlockSpec((30,), lambda i: (0,)),
        ],
        out_specs=pl.BlockSpec((_BNC, 30), lambda i: (i, 0)),
        out_shape=jax.ShapeDtypeStruct((N, 30), jnp.float32),
    )(u2p, hs2p, W2, b2, Wf1, bf1, Wf2, bf2, Wo, bo)


def kernel(x, edge_index, globf, W1, b1, W2, b2, Wf1, bf1, Wf2, bf2, Wo, bo):
    ei = edge_index.astype(jnp.int32)
    pad_tgt = N + (jnp.arange(EP - E, dtype=jnp.int32) % NPAD)
    src = jnp.concatenate([ei[0], pad_tgt])
    dst = jnp.concatenate([ei[1], pad_tgt])
    ed = jnp.stack([src * 8, dst])
    z1 = jnp.zeros((NP,), jnp.float32)

    dg0, dg1 = _sc_degree(dst, z1)
    dinv = lax.rsqrt(dg0 + dg1 + 1.0)
    x5 = jnp.pad(x, ((0, NPAD), (0, 0)))
    hs1p = jnp.concatenate(
        [dinv[:, None] * x5, jnp.zeros((NP, 3), jnp.float32),
         dinv[:, None], jnp.zeros((NP, 119), jnp.float32)], axis=1)

    u1p = _sc_agg(_agg1_body, ed, hs1p.reshape(8 * NP, 16))
    gfp = jnp.pad(globf, ((0, NPAD), (0, 0)))
    hs2p = _tcB(u1p, hs1p, gfp, W1, b1)

    u2p = _sc_agg(_agg2_body, ed, hs2p.reshape(8 * NP, 16))
    return _tcC(u2p, hs2p, W2, b2, Wf1, bf1, Wf2, bf2, Wo, bo)
